# Initial kernel scaffold; baseline (speedup 1.0000x reference)
import functools

import jax
import jax.numpy as jnp
from jax import lax
from jax.experimental import pallas as pl
from jax.experimental.pallas import tpu as pltpu
from jax.experimental.pallas import tpu_sc as plsc

N = 10000
E = 320000
D = 128

NC = 2
NS = 16
NW = NC * NS
EPW = E // NW

CH = 40
NCHUNK = EPW // CH
NPAIR = NCHUNK // 2

CHD = 80
NCHUNK_D = EPW // CHD

RPT = 640
RPT_LAST = N - 15 * RPT


def _zero_rows_buf(buf, nrows):
    z = jnp.zeros((16,), jnp.float32)

    def body(i, _):
        r = i // (D // 16)
        j = i % (D // 16)
        buf[r, pl.ds(j * 16, 16)] = z
        return 0

    lax.fori_loop(0, nrows * (D // 16), body, 0)


def _sc_scatter_body(g_hbm, src_hbm, dst_hbm, out_hbm,
                     sidx, didx, rows0, rows1, acc, sem0, sem1):
    cid = lax.axis_index("c")
    sid = lax.axis_index("s")
    wid = sid * NC + cid

    _zero_rows_buf(rows0, CH)

    def zbody(k, _):
        start = sid * RPT + k * CH

        @pl.when(start < N)
        def _():
            pltpu.sync_copy(rows0, acc.at[pl.ds(start, CH)])
        return 0

    lax.fori_loop(0, RPT // CH, zbody, 0)

    row0 = wid * NCHUNK
    pltpu.sync_copy(src_hbm.at[pl.ds(row0, NCHUNK)], sidx)
    pltpu.sync_copy(dst_hbm.at[pl.ds(row0, NCHUNK)], didx)

    plsc.subcore_barrier()

    pltpu.async_copy(g_hbm.at[sidx.at[0]], rows0, sem0)

    def pbody(i, _):
        c0 = i * 2
        c1 = c0 + 1
        pltpu.make_async_copy(g_hbm.at[sidx.at[c0]], rows0, sem0).wait()
        pltpu.async_copy(g_hbm.at[sidx.at[c1]], rows1, sem1)
        pltpu.sync_copy(rows0, acc.at[didx.at[c0]], add=True)
        pltpu.make_async_copy(g_hbm.at[sidx.at[c1]], rows1, sem1).wait()

        @pl.when(i < NPAIR - 1)
        def _():
            pltpu.async_copy(g_hbm.at[sidx.at[c0 + 2]], rows0, sem0)

        pltpu.sync_copy(rows1, acc.at[didx.at[c1]], add=True)
        return 0

    lax.fori_loop(0, NPAIR, pbody, 0)

    plsc.subcore_barrier()

    @pl.when(sid < 15)
    def _():
        pltpu.sync_copy(acc.at[pl.ds(sid * RPT, RPT)],
                        out_hbm.at[cid, pl.ds(sid * RPT, RPT)])

    @pl.when(sid == 15)
    def _():
        pltpu.sync_copy(acc.at[pl.ds(15 * RPT, RPT_LAST)],
                        out_hbm.at[cid, pl.ds(15 * RPT, RPT_LAST)])


def _sc_scatter(g, src2d, dst2d):
    mesh = plsc.VectorSubcoreMesh(core_axis_name="c", subcore_axis_name="s")
    return pl.kernel(
        _sc_scatter_body,
        out_type=jax.ShapeDtypeStruct((NC, N, D), jnp.float32),
        mesh=mesh,
        scratch_types=[
            pltpu.VMEM((NCHUNK, CH), jnp.int32),
            pltpu.VMEM((NCHUNK, CH), jnp.int32),
            pltpu.VMEM((CH, D), jnp.float32),
            pltpu.VMEM((CH, D), jnp.float32),
            pltpu.VMEM_SHARED((N, D), jnp.float32),
            pltpu.SemaphoreType.DMA,
            pltpu.SemaphoreType.DMA,
        ],
    )(g, src2d, dst2d)


def _sc_deg_body(dst_hbm, deg_hbm, didx, ones_v, zbuf, acc):
    cid = lax.axis_index("c")
    sid = lax.axis_index("s")
    wid = sid * NC + cid

    z = jnp.zeros((16,), jnp.float32)
    o = jnp.ones((16,), jnp.float32)
    for i in range(CHD // 16):
        zbuf[pl.ds(i * 16, 16)] = z
        ones_v[pl.ds(i * 16, 16)] = o

    def zbody(k, _):
        start = sid * RPT + k * CHD

        @pl.when(start < N)
        def _():
            pltpu.sync_copy(zbuf, acc.at[pl.ds(start, CHD)])
        return 0

    lax.fori_loop(0, RPT // CHD, zbody, 0)

    row0 = wid * NCHUNK_D
    pltpu.sync_copy(dst_hbm.at[pl.ds(row0, NCHUNK_D)], didx)

    plsc.subcore_barrier()

    def body(c, _):
        pltpu.sync_copy(ones_v, acc.at[didx.at[c]], add=True)
        return 0

    lax.fori_loop(0, NCHUNK_D, body, 0)

    plsc.subcore_barrier()

    @pl.when(sid < 15)
    def _():
        pltpu.sync_copy(acc.at[pl.ds(sid * RPT, RPT)],
                        deg_hbm.at[cid, pl.ds(sid * RPT, RPT)])

    @pl.when(sid == 15)
    def _():
        pltpu.sync_copy(acc.at[pl.ds(15 * RPT, RPT_LAST)],
                        deg_hbm.at[cid, pl.ds(15 * RPT, RPT_LAST)])


def _sc_deg(dst2d_deg):
    mesh = plsc.VectorSubcoreMesh(core_axis_name="c", subcore_axis_name="s")
    return pl.kernel(
        _sc_deg_body,
        out_type=jax.ShapeDtypeStruct((NC, N), jnp.float32),
        mesh=mesh,
        scratch_types=[
            pltpu.VMEM((NCHUNK_D, CHD), jnp.int32),
            pltpu.VMEM((CHD,), jnp.float32),
            pltpu.VMEM((CHD,), jnp.float32),
            pltpu.VMEM_SHARED((N,), jnp.float32),
        ],
    )(dst2d_deg)


BR = 1000
GRID = N // BR


def _dinv(d0, d1):
    return lax.rsqrt(d0 + d1 + 1.0)


def _k1_body(x_ref, w_ref, d0_ref, d1_ref, g_ref):
    dinv = _dinv(d0_ref[...], d1_ref[...])
    g_ref[...] = jnp.dot(x_ref[...], w_ref[...],
                         preferred_element_type=jnp.float32) * dinv


def _tc_k1(x, W1, d0, d1):
    return pl.pallas_call(
        _k1_body,
        grid=(GRID,),
        in_specs=[
            pl.BlockSpec((BR, D), lambda i: (i, 0)),
            pl.BlockSpec((D, D), lambda i: (0, 0)),
            pl.BlockSpec((BR, 1), lambda i: (i, 0)),
            pl.BlockSpec((BR, 1), lambda i: (i, 0)),
        ],
        out_specs=pl.BlockSpec((BR, D), lambda i: (i, 0)),
        out_shape=jax.ShapeDtypeStruct((N, D), jnp.float32),
    )(x, W1, d0, d1)


def _k2_body(p0_ref, p1_ref, g1_ref, d0_ref, d1_ref, b1_ref, w2_ref, g2_ref):
    dinv = _dinv(d0_ref[...], d1_ref[...])
    h = dinv * (p0_ref[...] + p1_ref[...] + g1_ref[...]) + b1_ref[...]
    h = jnp.maximum(h, 0.0)
    g2_ref[...] = jnp.dot(h, w2_ref[...],
                          preferred_element_type=jnp.float32) * dinv


def _tc_k2(p0, p1, g1, d0, d1, b1, W2):
    return pl.pallas_call(
        _k2_body,
        grid=(GRID,),
        in_specs=[
            pl.BlockSpec((BR, D), lambda i: (i, 0)),
            pl.BlockSpec((BR, D), lambda i: (i, 0)),
            pl.BlockSpec((BR, D), lambda i: (i, 0)),
            pl.BlockSpec((BR, 1), lambda i: (i, 0)),
            pl.BlockSpec((BR, 1), lambda i: (i, 0)),
            pl.BlockSpec((1, D), lambda i: (0, 0)),
            pl.BlockSpec((D, D), lambda i: (0, 0)),
        ],
        out_specs=pl.BlockSpec((BR, D), lambda i: (i, 0)),
        out_shape=jax.ShapeDtypeStruct((N, D), jnp.float32),
    )(p0, p1, g1, d0, d1, b1, W2)


def _k3_body(p0_ref, p1_ref, g2_ref, d0_ref, d1_ref, b2_ref, z_ref):
    dinv = _dinv(d0_ref[...], d1_ref[...])
    z_ref[...] = dinv * (p0_ref[...] + p1_ref[...] + g2_ref[...]) + b2_ref[...]


def _tc_k3(p0, p1, g2, d0, d1, b2):
    return pl.pallas_call(
        _k3_body,
        grid=(GRID,),
        in_specs=[
            pl.BlockSpec((BR, D), lambda i: (i, 0)),
            pl.BlockSpec((BR, D), lambda i: (i, 0)),
            pl.BlockSpec((BR, D), lambda i: (i, 0)),
            pl.BlockSpec((BR, 1), lambda i: (i, 0)),
            pl.BlockSpec((BR, 1), lambda i: (i, 0)),
            pl.BlockSpec((1, D), lambda i: (0, 0)),
        ],
        out_specs=pl.BlockSpec((BR, D), lambda i: (i, 0)),
        out_shape=jax.ShapeDtypeStruct((N, D), jnp.float32),
    )(p0, p1, g2, d0, d1, b2)


def kernel(x, edge_index, W1, b1, W2, b2):
    src = edge_index[0]
    dst = edge_index[1]
    src2d = src.reshape(E // CH, CH)
    dst2d = dst.reshape(E // CH, CH)
    dst2d_deg = dst.reshape(E // CHD, CHD)

    degp = _sc_deg(dst2d_deg)
    d0 = degp[0].reshape(N, 1)
    d1 = degp[1].reshape(N, 1)

    g1 = _tc_k1(x, W1, d0, d1)
    s1 = _sc_scatter(g1, src2d, dst2d)
    g2 = _tc_k2(s1[0], s1[1], g1, d0, d1, b1.reshape(1, D), W2)
    s2 = _sc_scatter(g2, src2d, dst2d)
    z = _tc_k3(s2[0], s2[1], g2, d0, d1, b2.reshape(1, D))
    return z


# speedup vs baseline: 9.7751x; 9.7751x over previous
import jax
import jax.numpy as jnp
from jax import lax
from jax.experimental import pallas as pl
from jax.experimental.pallas import tpu as pltpu
from jax.experimental.pallas import tpu_sc as plsc

N = 10000
E = 320000
D = 128

NC = 2
NS = 16
NW = NC * NS

CH = 128
NCH_W = 80
EP = NW * NCH_W * CH
HALF = NCH_W // 2
NPAIR = HALF // 2

NP = 10240
RPT = NP // NS


def _zero_rows_buf(buf, nrows):
    z = jnp.zeros((16,), jnp.float32)

    def body(i, _):
        r = i // (D // 16)
        j = i % (D // 16)
        buf[r, pl.ds(j * 16, 16)] = z
        return 0

    lax.fori_loop(0, nrows * (D // 16), body, 0)


def _sc_scatter_body(g_hbm, src_hbm, dst_hbm, out_hbm,
                     sidx, didx, rows0, rows1, acc, sem0, sem1):
    cid = lax.axis_index("c")
    sid = lax.axis_index("s")
    wid = sid * NC + cid

    _zero_rows_buf(rows0, CH)

    def zbody(k, _):
        pltpu.sync_copy(rows0, acc.at[pl.ds(sid * RPT + k * CH, CH)])
        return 0

    lax.fori_loop(0, RPT // CH, zbody, 0)

    plsc.subcore_barrier()

    for h in range(2):
        row0 = wid * NCH_W + h * HALF
        pltpu.sync_copy(src_hbm.at[pl.ds(row0, HALF)], sidx)
        pltpu.sync_copy(dst_hbm.at[pl.ds(row0, HALF)], didx)

        pltpu.async_copy(g_hbm.at[sidx.at[0]], rows0, sem0)

        def pbody(i, _):
            c0 = i * 2
            c1 = c0 + 1
            pltpu.make_async_copy(g_hbm.at[sidx.at[c0]], rows0, sem0).wait()
            pltpu.async_copy(g_hbm.at[sidx.at[c1]], rows1, sem1)
            pltpu.sync_copy(rows0, acc.at[didx.at[c0]], add=True)
            pltpu.make_async_copy(g_hbm.at[sidx.at[c1]], rows1, sem1).wait()

            @pl.when(i < NPAIR - 1)
            def _():
                pltpu.async_copy(g_hbm.at[sidx.at[c0 + 2]], rows0, sem0)

            pltpu.sync_copy(rows1, acc.at[didx.at[c1]], add=True)
            return 0

        lax.fori_loop(0, NPAIR, pbody, 0)

    plsc.subcore_barrier()

    pltpu.sync_copy(acc.at[pl.ds(sid * RPT, RPT)],
                    out_hbm.at[cid, pl.ds(sid * RPT, RPT)])


def _sc_scatter(g, src2d, dst2d):
    mesh = plsc.VectorSubcoreMesh(core_axis_name="c", subcore_axis_name="s")
    return pl.kernel(
        _sc_scatter_body,
        out_type=jax.ShapeDtypeStruct((NC, NP, D), jnp.float32),
        mesh=mesh,
        scratch_types=[
            pltpu.VMEM((HALF, CH), jnp.int32),
            pltpu.VMEM((HALF, CH), jnp.int32),
            pltpu.VMEM((CH, D), jnp.float32),
            pltpu.VMEM((CH, D), jnp.float32),
            pltpu.VMEM_SHARED((NP, D), jnp.float32),
            pltpu.SemaphoreType.DMA,
            pltpu.SemaphoreType.DMA,
        ],
    )(g, src2d, dst2d)


def _sc_deg_body(dst_hbm, deg_hbm, didx, ones_v, zbuf, acc):
    cid = lax.axis_index("c")
    sid = lax.axis_index("s")
    wid = sid * NC + cid

    z = jnp.zeros((16,), jnp.float32)
    o = jnp.ones((16,), jnp.float32)
    for i in range(CH // 16):
        zbuf[pl.ds(i * 16, 16)] = z
        ones_v[pl.ds(i * 16, 16)] = o

    def zbody(k, _):
        pltpu.sync_copy(zbuf, acc.at[pl.ds(sid * RPT + k * CH, CH)])
        return 0

    lax.fori_loop(0, RPT // CH, zbody, 0)

    pltpu.sync_copy(dst_hbm.at[pl.ds(wid * NCH_W, NCH_W)], didx)

    plsc.subcore_barrier()

    def body(c, _):
        pltpu.sync_copy(ones_v, acc.at[didx.at[c]], add=True)
        return 0

    lax.fori_loop(0, NCH_W, body, 0)

    plsc.subcore_barrier()

    pltpu.sync_copy(acc.at[pl.ds(sid * RPT, RPT)],
                    deg_hbm.at[cid, pl.ds(sid * RPT, RPT)])


def _sc_deg(dst2d):
    mesh = plsc.VectorSubcoreMesh(core_axis_name="c", subcore_axis_name="s")
    return pl.kernel(
        _sc_deg_body,
        out_type=jax.ShapeDtypeStruct((NC, NP), jnp.float32),
        mesh=mesh,
        scratch_types=[
            pltpu.VMEM((NCH_W, CH), jnp.int32),
            pltpu.VMEM((CH,), jnp.float32),
            pltpu.VMEM((CH,), jnp.float32),
            pltpu.VMEM_SHARED((NP,), jnp.float32),
        ],
    )(dst2d)


BR = 1000
GRID = N // BR


def _dinv(d0, d1):
    return lax.rsqrt(d0 + d1 + 1.0)


def _k1_body(x_ref, w_ref, d0_ref, d1_ref, g_ref):
    dinv = _dinv(d0_ref[...], d1_ref[...])
    g_ref[...] = jnp.dot(x_ref[...], w_ref[...],
                         preferred_element_type=jnp.float32) * dinv


def _tc_k1(x, W1, d0, d1):
    return pl.pallas_call(
        _k1_body,
        grid=(GRID,),
        in_specs=[
            pl.BlockSpec((BR, D), lambda i: (i, 0)),
            pl.BlockSpec((D, D), lambda i: (0, 0)),
            pl.BlockSpec((BR, 1), lambda i: (i, 0)),
            pl.BlockSpec((BR, 1), lambda i: (i, 0)),
        ],
        out_specs=pl.BlockSpec((BR, D), lambda i: (i, 0)),
        out_shape=jax.ShapeDtypeStruct((N, D), jnp.float32),
    )(x, W1, d0, d1)


def _k2_body(p0_ref, p1_ref, g1_ref, d0_ref, d1_ref, b1_ref, w2_ref, g2_ref):
    dinv = _dinv(d0_ref[...], d1_ref[...])
    h = dinv * (p0_ref[...] + p1_ref[...] + g1_ref[...]) + b1_ref[...]
    h = jnp.maximum(h, 0.0)
    g2_ref[...] = jnp.dot(h, w2_ref[...],
                          preferred_element_type=jnp.float32) * dinv


def _tc_k2(p0, p1, g1, d0, d1, b1, W2):
    return pl.pallas_call(
        _k2_body,
        grid=(GRID,),
        in_specs=[
            pl.BlockSpec((BR, D), lambda i: (i, 0)),
            pl.BlockSpec((BR, D), lambda i: (i, 0)),
            pl.BlockSpec((BR, D), lambda i: (i, 0)),
            pl.BlockSpec((BR, 1), lambda i: (i, 0)),
            pl.BlockSpec((BR, 1), lambda i: (i, 0)),
            pl.BlockSpec((1, D), lambda i: (0, 0)),
            pl.BlockSpec((D, D), lambda i: (0, 0)),
        ],
        out_specs=pl.BlockSpec((BR, D), lambda i: (i, 0)),
        out_shape=jax.ShapeDtypeStruct((N, D), jnp.float32),
    )(p0, p1, g1, d0, d1, b1, W2)


def _k3_body(p0_ref, p1_ref, g2_ref, d0_ref, d1_ref, b2_ref, z_ref):
    dinv = _dinv(d0_ref[...], d1_ref[...])
    z_ref[...] = dinv * (p0_ref[...] + p1_ref[...] + g2_ref[...]) + b2_ref[...]


def _tc_k3(p0, p1, g2, d0, d1, b2):
    return pl.pallas_call(
        _k3_body,
        grid=(GRID,),
        in_specs=[
            pl.BlockSpec((BR, D), lambda i: (i, 0)),
            pl.BlockSpec((BR, D), lambda i: (i, 0)),
            pl.BlockSpec((BR, D), lambda i: (i, 0)),
            pl.BlockSpec((BR, 1), lambda i: (i, 0)),
            pl.BlockSpec((BR, 1), lambda i: (i, 0)),
            pl.BlockSpec((1, D), lambda i: (0, 0)),
        ],
        out_specs=pl.BlockSpec((BR, D), lambda i: (i, 0)),
        out_shape=jax.ShapeDtypeStruct((N, D), jnp.float32),
    )(p0, p1, g2, d0, d1, b2)


def kernel(x, edge_index, W1, b1, W2, b2):
    src = edge_index[0]
    dst = edge_index[1]
    pad = EP - E
    src2d = jnp.concatenate(
        [src, jnp.zeros((pad,), jnp.int32)]).reshape(EP // CH, CH)
    dst2d = jnp.concatenate(
        [dst, jnp.full((pad,), NP - 1, jnp.int32)]).reshape(EP // CH, CH)

    degp = _sc_deg(dst2d)
    d0 = degp[0, :N].reshape(N, 1)
    d1 = degp[1, :N].reshape(N, 1)

    g1 = _tc_k1(x, W1, d0, d1)
    s1 = _sc_scatter(g1, src2d, dst2d)
    g2 = _tc_k2(s1[0, :N], s1[1, :N], g1, d0, d1, b1.reshape(1, D), W2)
    s2 = _sc_scatter(g2, src2d, dst2d)
    z = _tc_k3(s2[0, :N], s2[1, :N], g2, d0, d1, b2.reshape(1, D))
    return z


# speedup vs baseline: 10.9873x; 1.1240x over previous
import jax
import jax.numpy as jnp
from jax import lax
from jax.experimental import pallas as pl
from jax.experimental.pallas import tpu as pltpu
from jax.experimental.pallas import tpu_sc as plsc

N = 10000
E = 320000
D = 128

NC = 2
NS = 16
NW = NC * NS

CH = 128
NCH_W = 80
EP = NW * NCH_W * CH

C_FAST = 0
NQ = 4
QF = 32
QS = 8
PAIR_CH = NQ * (QF + QS)

NP = 10240
RPT = NP // NS


def _zero_rows_buf(buf, nrows):
    z = jnp.zeros((16,), jnp.float32)

    def body(i, _):
        r = i // (D // 16)
        j = i % (D // 16)
        buf[r, pl.ds(j * 16, 16)] = z
        return 0

    lax.fori_loop(0, nrows * (D // 16), body, 0)


def _sc_scatter_body(g_hbm, src_hbm, dst_hbm, out_hbm,
                     sidx, didx, rows0, rows1, acc, sem0, sem1):
    cid = lax.axis_index("c")
    sid = lax.axis_index("s")
    wid = sid * NC + cid

    _zero_rows_buf(rows0, CH)

    def zbody(k, _):
        pltpu.sync_copy(rows0, acc.at[pl.ds(sid * RPT + k * CH, CH)])
        return 0

    lax.fori_loop(0, RPT // CH, zbody, 0)

    plsc.subcore_barrier()

    is_fast = cid == C_FAST
    mypairs = jnp.where(is_fast, QF // 2, QS // 2)
    mybase = sid * PAIR_CH + jnp.where(is_fast, 0, NQ * QF)

    for q in range(NQ):
        @pl.when(is_fast)
        def _():
            pltpu.sync_copy(src_hbm.at[pl.ds(mybase + q * QF, QF)],
                            sidx.at[pl.ds(0, QF)])
            pltpu.sync_copy(dst_hbm.at[pl.ds(mybase + q * QF, QF)],
                            didx.at[pl.ds(0, QF)])

        @pl.when(jnp.logical_not(is_fast))
        def _():
            pltpu.sync_copy(src_hbm.at[pl.ds(mybase + q * QS, QS)],
                            sidx.at[pl.ds(0, QS)])
            pltpu.sync_copy(dst_hbm.at[pl.ds(mybase + q * QS, QS)],
                            didx.at[pl.ds(0, QS)])

        pltpu.async_copy(g_hbm.at[sidx.at[0]], rows0, sem0)

        def pbody(i, _):
            c0 = i * 2
            c1 = c0 + 1
            pltpu.make_async_copy(g_hbm.at[sidx.at[c0]], rows0, sem0).wait()
            pltpu.async_copy(g_hbm.at[sidx.at[c1]], rows1, sem1)
            pltpu.sync_copy(rows0, acc.at[didx.at[c0]], add=True)
            pltpu.make_async_copy(g_hbm.at[sidx.at[c1]], rows1, sem1).wait()

            @pl.when(i < mypairs - 1)
            def _():
                pltpu.async_copy(g_hbm.at[sidx.at[c0 + 2]], rows0, sem0)

            pltpu.sync_copy(rows1, acc.at[didx.at[c1]], add=True)
            return 0

        lax.fori_loop(0, mypairs, pbody, 0)

    plsc.subcore_barrier()

    pltpu.sync_copy(acc.at[pl.ds(sid * RPT, RPT)],
                    out_hbm.at[cid, pl.ds(sid * RPT, RPT)])


def _sc_scatter(g, src2d, dst2d):
    mesh = plsc.VectorSubcoreMesh(core_axis_name="c", subcore_axis_name="s")
    return pl.kernel(
        _sc_scatter_body,
        out_type=jax.ShapeDtypeStruct((NC, NP, D), jnp.float32),
        mesh=mesh,
        scratch_types=[
            pltpu.VMEM((QF, CH), jnp.int32),
            pltpu.VMEM((QF, CH), jnp.int32),
            pltpu.VMEM((CH, D), jnp.float32),
            pltpu.VMEM((CH, D), jnp.float32),
            pltpu.VMEM_SHARED((NP, D), jnp.float32),
            pltpu.SemaphoreType.DMA,
            pltpu.SemaphoreType.DMA,
        ],
    )(g, src2d, dst2d)


def _sc_deg_body(dst_hbm, deg_hbm, didx, ones_v, zbuf, acc):
    cid = lax.axis_index("c")
    sid = lax.axis_index("s")
    wid = sid * NC + cid

    z = jnp.zeros((16,), jnp.float32)
    o = jnp.ones((16,), jnp.float32)
    for i in range(CH // 16):
        zbuf[pl.ds(i * 16, 16)] = z
        ones_v[pl.ds(i * 16, 16)] = o

    def zbody(k, _):
        pltpu.sync_copy(zbuf, acc.at[pl.ds(sid * RPT + k * CH, CH)])
        return 0

    lax.fori_loop(0, RPT // CH, zbody, 0)

    pltpu.sync_copy(dst_hbm.at[pl.ds(wid * NCH_W, NCH_W)], didx)

    plsc.subcore_barrier()

    def body(c, _):
        pltpu.sync_copy(ones_v, acc.at[didx.at[c]], add=True)
        return 0

    lax.fori_loop(0, NCH_W, body, 0)

    plsc.subcore_barrier()

    pltpu.sync_copy(acc.at[pl.ds(sid * RPT, RPT)],
                    deg_hbm.at[cid, pl.ds(sid * RPT, RPT)])


def _sc_deg(dst2d):
    mesh = plsc.VectorSubcoreMesh(core_axis_name="c", subcore_axis_name="s")
    return pl.kernel(
        _sc_deg_body,
        out_type=jax.ShapeDtypeStruct((NC, NP), jnp.float32),
        mesh=mesh,
        scratch_types=[
            pltpu.VMEM((NCH_W, CH), jnp.int32),
            pltpu.VMEM((CH,), jnp.float32),
            pltpu.VMEM((CH,), jnp.float32),
            pltpu.VMEM_SHARED((NP,), jnp.float32),
        ],
    )(dst2d)


BR = 1000
GRID = N // BR


def _dinv(d0, d1):
    return lax.rsqrt(d0 + d1 + 1.0)


def _k1_body(x_ref, w_ref, d0_ref, d1_ref, g_ref):
    dinv = _dinv(d0_ref[...], d1_ref[...])
    g_ref[...] = jnp.dot(x_ref[...], w_ref[...],
                         preferred_element_type=jnp.float32) * dinv


def _tc_k1(x, W1, d0, d1):
    return pl.pallas_call(
        _k1_body,
        grid=(GRID,),
        in_specs=[
            pl.BlockSpec((BR, D), lambda i: (i, 0)),
            pl.BlockSpec((D, D), lambda i: (0, 0)),
            pl.BlockSpec((BR, 1), lambda i: (i, 0)),
            pl.BlockSpec((BR, 1), lambda i: (i, 0)),
        ],
        out_specs=pl.BlockSpec((BR, D), lambda i: (i, 0)),
        out_shape=jax.ShapeDtypeStruct((N, D), jnp.float32),
    )(x, W1, d0, d1)


def _k2_body(p0_ref, p1_ref, g1_ref, d0_ref, d1_ref, b1_ref, w2_ref, g2_ref):
    dinv = _dinv(d0_ref[...], d1_ref[...])
    h = dinv * (p0_ref[...] + p1_ref[...] + g1_ref[...]) + b1_ref[...]
    h = jnp.maximum(h, 0.0)
    g2_ref[...] = jnp.dot(h, w2_ref[...],
                          preferred_element_type=jnp.float32) * dinv


def _tc_k2(p0, p1, g1, d0, d1, b1, W2):
    return pl.pallas_call(
        _k2_body,
        grid=(GRID,),
        in_specs=[
            pl.BlockSpec((BR, D), lambda i: (i, 0)),
            pl.BlockSpec((BR, D), lambda i: (i, 0)),
            pl.BlockSpec((BR, D), lambda i: (i, 0)),
            pl.BlockSpec((BR, 1), lambda i: (i, 0)),
            pl.BlockSpec((BR, 1), lambda i: (i, 0)),
            pl.BlockSpec((1, D), lambda i: (0, 0)),
            pl.BlockSpec((D, D), lambda i: (0, 0)),
        ],
        out_specs=pl.BlockSpec((BR, D), lambda i: (i, 0)),
        out_shape=jax.ShapeDtypeStruct((N, D), jnp.float32),
    )(p0, p1, g1, d0, d1, b1, W2)


def _k3_body(p0_ref, p1_ref, g2_ref, d0_ref, d1_ref, b2_ref, z_ref):
    dinv = _dinv(d0_ref[...], d1_ref[...])
    z_ref[...] = dinv * (p0_ref[...] + p1_ref[...] + g2_ref[...]) + b2_ref[...]


def _tc_k3(p0, p1, g2, d0, d1, b2):
    return pl.pallas_call(
        _k3_body,
        grid=(GRID,),
        in_specs=[
            pl.BlockSpec((BR, D), lambda i: (i, 0)),
            pl.BlockSpec((BR, D), lambda i: (i, 0)),
            pl.BlockSpec((BR, D), lambda i: (i, 0)),
            pl.BlockSpec((BR, 1), lambda i: (i, 0)),
            pl.BlockSpec((BR, 1), lambda i: (i, 0)),
            pl.BlockSpec((1, D), lambda i: (0, 0)),
        ],
        out_specs=pl.BlockSpec((BR, D), lambda i: (i, 0)),
        out_shape=jax.ShapeDtypeStruct((N, D), jnp.float32),
    )(p0, p1, g2, d0, d1, b2)


def kernel(x, edge_index, W1, b1, W2, b2):
    src = edge_index[0]
    dst = edge_index[1]
    pad = EP - E
    src2d = jnp.concatenate(
        [src, jnp.zeros((pad,), jnp.int32)]).reshape(EP // CH, CH)
    dst2d = jnp.concatenate(
        [dst, jnp.full((pad,), NP - 1, jnp.int32)]).reshape(EP // CH, CH)

    degp = _sc_deg(dst2d)
    d0 = degp[0, :N].reshape(N, 1)
    d1 = degp[1, :N].reshape(N, 1)

    g1 = _tc_k1(x, W1, d0, d1)
    s1 = _sc_scatter(g1, src2d, dst2d)
    g2 = _tc_k2(s1[0, :N], s1[1, :N], g1, d0, d1, b1.reshape(1, D), W2)
    s2 = _sc_scatter(g2, src2d, dst2d)
    z = _tc_k3(s2[0, :N], s2[1, :N], g2, d0, d1, b2.reshape(1, D))
    return z


# speedup vs baseline: 10.9895x; 1.0002x over previous
import jax
import jax.numpy as jnp
from jax import lax
from jax.experimental import pallas as pl
from jax.experimental.pallas import tpu as pltpu
from jax.experimental.pallas import tpu_sc as plsc

N = 10000
E = 320000
D = 128

NC = 2
NS = 16
NW = NC * NS

CH = 128
NCH_W = 80
EP = NW * NCH_W * CH

C_FAST = 0
NQ = 2
QF = 56
QS = 24
PAIR_CH = NQ * (QF + QS)

NP = 10240
RPT = NP // NS


def _zero_rows_buf(buf, nrows):
    z = jnp.zeros((16,), jnp.float32)

    def body(i, _):
        r = i // (D // 16)
        j = i % (D // 16)
        buf[r, pl.ds(j * 16, 16)] = z
        return 0

    lax.fori_loop(0, nrows * (D // 16), body, 0)


def _sc_scatter_body(g_hbm, src_hbm, dst_hbm, out_hbm,
                     sidx, didx, rows0, rows1, acc, sem0, sem1):
    cid = lax.axis_index("c")
    sid = lax.axis_index("s")
    wid = sid * NC + cid

    _zero_rows_buf(rows0, CH)

    def zbody(k, _):
        pltpu.sync_copy(rows0, acc.at[pl.ds(sid * RPT + k * CH, CH)])
        return 0

    lax.fori_loop(0, RPT // CH, zbody, 0)

    plsc.subcore_barrier()

    is_fast = cid == C_FAST
    mypairs = jnp.where(is_fast, QF // 2, QS // 2)
    mybase = sid * PAIR_CH + jnp.where(is_fast, 0, NQ * QF)

    for q in range(NQ):
        @pl.when(is_fast)
        def _():
            pltpu.sync_copy(src_hbm.at[pl.ds(mybase + q * QF, QF)],
                            sidx.at[pl.ds(0, QF)])
            pltpu.sync_copy(dst_hbm.at[pl.ds(mybase + q * QF, QF)],
                            didx.at[pl.ds(0, QF)])

        @pl.when(jnp.logical_not(is_fast))
        def _():
            pltpu.sync_copy(src_hbm.at[pl.ds(mybase + q * QS, QS)],
                            sidx.at[pl.ds(0, QS)])
            pltpu.sync_copy(dst_hbm.at[pl.ds(mybase + q * QS, QS)],
                            didx.at[pl.ds(0, QS)])

        pltpu.async_copy(g_hbm.at[sidx.at[0]], rows0, sem0)

        def pbody(i, _):
            c0 = i * 2
            c1 = c0 + 1
            pltpu.make_async_copy(g_hbm.at[sidx.at[c0]], rows0, sem0).wait()
            pltpu.async_copy(g_hbm.at[sidx.at[c1]], rows1, sem1)
            pltpu.sync_copy(rows0, acc.at[didx.at[c0]], add=True)
            pltpu.make_async_copy(g_hbm.at[sidx.at[c1]], rows1, sem1).wait()

            @pl.when(i < mypairs - 1)
            def _():
                pltpu.async_copy(g_hbm.at[sidx.at[c0 + 2]], rows0, sem0)

            pltpu.sync_copy(rows1, acc.at[didx.at[c1]], add=True)
            return 0

        lax.fori_loop(0, mypairs, pbody, 0)

    plsc.subcore_barrier()

    pltpu.sync_copy(acc.at[pl.ds(sid * RPT, RPT)],
                    out_hbm.at[cid, pl.ds(sid * RPT, RPT)])


def _sc_scatter(g, src2d, dst2d):
    mesh = plsc.VectorSubcoreMesh(core_axis_name="c", subcore_axis_name="s")
    return pl.kernel(
        _sc_scatter_body,
        out_type=jax.ShapeDtypeStruct((NC, NP, D), jnp.float32),
        mesh=mesh,
        scratch_types=[
            pltpu.VMEM((QF, CH), jnp.int32),
            pltpu.VMEM((QF, CH), jnp.int32),
            pltpu.VMEM((CH, D), jnp.float32),
            pltpu.VMEM((CH, D), jnp.float32),
            pltpu.VMEM_SHARED((NP, D), jnp.float32),
            pltpu.SemaphoreType.DMA,
            pltpu.SemaphoreType.DMA,
        ],
    )(g, src2d, dst2d)


def _sc_deg_body(dst_hbm, deg_hbm, didx, ones_v, zbuf, acc):
    cid = lax.axis_index("c")
    sid = lax.axis_index("s")
    wid = sid * NC + cid

    z = jnp.zeros((16,), jnp.float32)
    o = jnp.ones((16,), jnp.float32)
    for i in range(CH // 16):
        zbuf[pl.ds(i * 16, 16)] = z
        ones_v[pl.ds(i * 16, 16)] = o

    def zbody(k, _):
        pltpu.sync_copy(zbuf, acc.at[pl.ds(sid * RPT + k * CH, CH)])
        return 0

    lax.fori_loop(0, RPT // CH, zbody, 0)

    pltpu.sync_copy(dst_hbm.at[pl.ds(wid * NCH_W, NCH_W)], didx)

    plsc.subcore_barrier()

    def body(c, _):
        pltpu.sync_copy(ones_v, acc.at[didx.at[c]], add=True)
        return 0

    lax.fori_loop(0, NCH_W, body, 0)

    plsc.subcore_barrier()

    pltpu.sync_copy(acc.at[pl.ds(sid * RPT, RPT)],
                    deg_hbm.at[cid, pl.ds(sid * RPT, RPT)])


def _sc_deg(dst2d):
    mesh = plsc.VectorSubcoreMesh(core_axis_name="c", subcore_axis_name="s")
    return pl.kernel(
        _sc_deg_body,
        out_type=jax.ShapeDtypeStruct((NC, NP), jnp.float32),
        mesh=mesh,
        scratch_types=[
            pltpu.VMEM((NCH_W, CH), jnp.int32),
            pltpu.VMEM((CH,), jnp.float32),
            pltpu.VMEM((CH,), jnp.float32),
            pltpu.VMEM_SHARED((NP,), jnp.float32),
        ],
    )(dst2d)


BR = 1000
GRID = N // BR


def _dinv(d0, d1):
    return lax.rsqrt(d0 + d1 + 1.0)


def _k1_body(x_ref, w_ref, d0_ref, d1_ref, g_ref):
    dinv = _dinv(d0_ref[...], d1_ref[...])
    g_ref[...] = jnp.dot(x_ref[...], w_ref[...],
                         preferred_element_type=jnp.float32) * dinv


def _tc_k1(x, W1, d0, d1):
    return pl.pallas_call(
        _k1_body,
        grid=(GRID,),
        in_specs=[
            pl.BlockSpec((BR, D), lambda i: (i, 0)),
            pl.BlockSpec((D, D), lambda i: (0, 0)),
            pl.BlockSpec((BR, 1), lambda i: (i, 0)),
            pl.BlockSpec((BR, 1), lambda i: (i, 0)),
        ],
        out_specs=pl.BlockSpec((BR, D), lambda i: (i, 0)),
        out_shape=jax.ShapeDtypeStruct((N, D), jnp.float32),
    )(x, W1, d0, d1)


def _k2_body(p0_ref, p1_ref, g1_ref, d0_ref, d1_ref, b1_ref, w2_ref, g2_ref):
    dinv = _dinv(d0_ref[...], d1_ref[...])
    h = dinv * (p0_ref[...] + p1_ref[...] + g1_ref[...]) + b1_ref[...]
    h = jnp.maximum(h, 0.0)
    g2_ref[...] = jnp.dot(h, w2_ref[...],
                          preferred_element_type=jnp.float32) * dinv


def _tc_k2(p0, p1, g1, d0, d1, b1, W2):
    return pl.pallas_call(
        _k2_body,
        grid=(GRID,),
        in_specs=[
            pl.BlockSpec((BR, D), lambda i: (i, 0)),
            pl.BlockSpec((BR, D), lambda i: (i, 0)),
            pl.BlockSpec((BR, D), lambda i: (i, 0)),
            pl.BlockSpec((BR, 1), lambda i: (i, 0)),
            pl.BlockSpec((BR, 1), lambda i: (i, 0)),
            pl.BlockSpec((1, D), lambda i: (0, 0)),
            pl.BlockSpec((D, D), lambda i: (0, 0)),
        ],
        out_specs=pl.BlockSpec((BR, D), lambda i: (i, 0)),
        out_shape=jax.ShapeDtypeStruct((N, D), jnp.float32),
    )(p0, p1, g1, d0, d1, b1, W2)


def _k3_body(p0_ref, p1_ref, g2_ref, d0_ref, d1_ref, b2_ref, z_ref):
    dinv = _dinv(d0_ref[...], d1_ref[...])
    z_ref[...] = dinv * (p0_ref[...] + p1_ref[...] + g2_ref[...]) + b2_ref[...]


def _tc_k3(p0, p1, g2, d0, d1, b2):
    return pl.pallas_call(
        _k3_body,
        grid=(GRID,),
        in_specs=[
            pl.BlockSpec((BR, D), lambda i: (i, 0)),
            pl.BlockSpec((BR, D), lambda i: (i, 0)),
            pl.BlockSpec((BR, D), lambda i: (i, 0)),
            pl.BlockSpec((BR, 1), lambda i: (i, 0)),
            pl.BlockSpec((BR, 1), lambda i: (i, 0)),
            pl.BlockSpec((1, D), lambda i: (0, 0)),
        ],
        out_specs=pl.BlockSpec((BR, D), lambda i: (i, 0)),
        out_shape=jax.ShapeDtypeStruct((N, D), jnp.float32),
    )(p0, p1, g2, d0, d1, b2)


def kernel(x, edge_index, W1, b1, W2, b2):
    src = edge_index[0]
    dst = edge_index[1]
    pad = EP - E
    src2d = jnp.concatenate(
        [src, jnp.zeros((pad,), jnp.int32)]).reshape(EP // CH, CH)
    dst2d = jnp.concatenate(
        [dst, jnp.full((pad,), NP - 1, jnp.int32)]).reshape(EP // CH, CH)

    degp = _sc_deg(dst2d)
    d0 = degp[0, :N].reshape(N, 1)
    d1 = degp[1, :N].reshape(N, 1)

    g1 = _tc_k1(x, W1, d0, d1)
    s1 = _sc_scatter(g1, src2d, dst2d)
    g2 = _tc_k2(s1[0, :N], s1[1, :N], g1, d0, d1, b1.reshape(1, D), W2)
    s2 = _sc_scatter(g2, src2d, dst2d)
    z = _tc_k3(s2[0, :N], s2[1, :N], g2, d0, d1, b2.reshape(1, D))
    return z
